# baseline (speedup 1.0000x reference)
import functools

import jax
import jax.numpy as jnp
from jax import lax
from jax.experimental import pallas as pl
from jax.experimental.pallas import tpu as pltpu
from jax.experimental.pallas import tpu_sc as plsc

_VOCAB = 100000
_DIM = 300
_PAD = 384
_BATCH = 16384

_NC = 2
_NS = 16
_NW = _NC * _NS
_CHUNK = 128
_CPW = _BATCH // (_NW * _CHUNK)
_BPW = _BATCH // _NW


def _embed_body(idx_hbm, tab_hbm, out_hbm, idx_v, buf0, buf1, sem0, sem1):
    wid = lax.axis_index("s") * _NC + lax.axis_index("c")
    pltpu.sync_copy(idx_hbm.at[pl.ds(wid * _BPW, _BPW)], idx_v)
    bufs = (buf0, buf1)
    sems = (sem0, sem1)

    def start(j):
        ii = idx_v.at[pl.ds(j * _CHUNK, _CHUNK)]
        return pltpu.async_copy(tab_hbm.at[ii], bufs[j % 2], sems[j % 2])

    copies = [start(0), None]
    for j in range(_CPW):
        b = j % 2
        if j + 1 < _CPW:
            copies[(j + 1) % 2] = start(j + 1)
        copies[b].wait()
        row0 = (wid * _CPW + j) * _CHUNK
        pltpu.sync_copy(bufs[b], out_hbm.at[pl.ds(row0, _CHUNK)])


_embed_lookup = functools.partial(
    pl.kernel,
    out_type=jax.ShapeDtypeStruct((_BATCH, _PAD), jnp.float32),
    mesh=plsc.VectorSubcoreMesh(core_axis_name="c", subcore_axis_name="s"),
    scratch_types=[
        pltpu.VMEM((_BPW,), jnp.int32),
        pltpu.VMEM((_CHUNK, _PAD), jnp.float32),
        pltpu.VMEM((_CHUNK, _PAD), jnp.float32),
        pltpu.SemaphoreType.DMA,
        pltpu.SemaphoreType.DMA,
    ],
)(_embed_body)


def kernel(tokens, table):
    idx = tokens.astype(jnp.int32)
    tabp = lax.pad(table, jnp.float32(0), [(0, 0, 0), (0, _PAD - _DIM, 0)])
    out_pad = _embed_lookup(idx, tabp)
    return out_pad[:, :_DIM]


# speedup vs baseline: 1.0001x; 1.0001x over previous
import functools

import jax
import jax.numpy as jnp
from jax import lax
from jax.experimental import pallas as pl
from jax.experimental.pallas import tpu as pltpu
from jax.experimental.pallas import tpu_sc as plsc

_VOCAB = 100000
_DIM = 300
_PAD = 384

_NC = 2
_NS = 16
_NW = _NC * _NS
_BATCH = 16384
_CHUNK = 128
_CPW = _BATCH // (_NW * _CHUNK)
_BPW = _BATCH // _NW


def _embed_body(idx_hbm, tab_hbm, out_hbm,
                idx_v, buf0, buf1, sem0, sem1):
    wid = lax.axis_index("s") * _NC + lax.axis_index("c")
    pltpu.sync_copy(idx_hbm.at[pl.ds(wid * _BPW, _BPW)], idx_v)
    bufs = (buf0, buf1)
    sems = (sem0, sem1)

    def start(j):
        b = j % 2
        ii = idx_v.at[pl.ds(j * _CHUNK, _CHUNK)]
        return pltpu.async_copy(tab_hbm.at[ii], bufs[b], sems[b])

    copies = [start(0), None]
    for j in range(_CPW):
        b = j % 2
        if j + 1 < _CPW:
            copies[(j + 1) % 2] = start(j + 1)
        copies[b].wait()
        rows = pl.ds((wid * _CPW + j) * _CHUNK, _CHUNK)
        pltpu.sync_copy(bufs[b], out_hbm.at[rows])


_embed_lookup = functools.partial(
    pl.kernel,
    out_type=jax.ShapeDtypeStruct((_BATCH, _PAD), jnp.float32),
    mesh=plsc.VectorSubcoreMesh(core_axis_name="c", subcore_axis_name="s"),
    scratch_types=[
        pltpu.VMEM((_BPW,), jnp.int32),
        pltpu.VMEM((_CHUNK, _PAD), jnp.float32),
        pltpu.VMEM((_CHUNK, _PAD), jnp.float32),
        pltpu.SemaphoreType.DMA,
        pltpu.SemaphoreType.DMA,
    ],
)(_embed_body)


def kernel(tokens, table):
    idx = tokens.astype(jnp.int32)
    padded = lax.pad(table, jnp.float32(0), [(0, 0, 0), (0, _PAD - _DIM, 0)])
    out_pad = _embed_lookup(idx, padded)
    return out_pad[:, :_DIM]


# speedup vs baseline: 1.0005x; 1.0004x over previous
import functools

import jax
import jax.numpy as jnp
from jax import lax
from jax.experimental import pallas as pl
from jax.experimental.pallas import tpu as pltpu
from jax.experimental.pallas import tpu_sc as plsc

_VOCAB = 100000
_DIM = 300
_PAD = 384

_NC = 2
_NS = 16
_NW = _NC * _NS
_BATCH = 16384
_CHUNK = 128
_CPW = _BATCH // (_NW * _CHUNK)
_BPW = _BATCH // _NW


def _embed_body(idx_hbm, tab_hbm, out_hbm,
                idx_v, buf0, buf1, sem0, sem1):
    wid = lax.axis_index("s") * _NC + lax.axis_index("c")
    pltpu.sync_copy(idx_hbm.at[pl.ds(wid * _BPW, _BPW)], idx_v)
    bufs = (buf0, buf1)
    sems = (sem0, sem1)

    def start(j):
        b = j % 2
        ii = idx_v.at[pl.ds(j * _CHUNK, _CHUNK)]
        return pltpu.async_copy(tab_hbm.at[ii], bufs[b], sems[b])

    copies = [start(0), None]
    for j in range(_CPW):
        b = j % 2
        if j + 1 < _CPW:
            copies[(j + 1) % 2] = start(j + 1)
        copies[b].wait()
        rows = pl.ds((wid * _CPW + j) * _CHUNK, _CHUNK)
        pltpu.sync_copy(bufs[b], out_hbm.at[rows])


_embed_lookup = functools.partial(
    pl.kernel,
    out_type=jax.ShapeDtypeStruct((_BATCH, _PAD), jnp.float32),
    mesh=plsc.VectorSubcoreMesh(core_axis_name="c", subcore_axis_name="s"),
    scratch_types=[
        pltpu.VMEM((_BPW,), jnp.int32),
        pltpu.VMEM((_CHUNK, _PAD), jnp.float32),
        pltpu.VMEM((_CHUNK, _PAD), jnp.float32),
        pltpu.SemaphoreType.DMA,
        pltpu.SemaphoreType.DMA,
    ],
)(_embed_body)


def kernel(tokens, table):
    idx = tokens.astype(jnp.int32)
    padded = lax.pad(table.T, jnp.float32(0),
                     [(0, _PAD - _DIM, 0), (0, 0, 0)]).T
    out_pad = _embed_lookup(idx, padded)
    return out_pad[:, :_DIM]
